# Initial kernel scaffold; baseline (speedup 1.0000x reference)
import jax
import jax.numpy as jnp
from jax.experimental import pallas as pl


def kernel(x, edge_index, W1, b1, W_skip02, b_skip02, W2, b2, W_skip13, b_skip13, W3, b3, W_lin, b_lin):
    raise NotImplementedError("write your pallas kernel here")


# speedup vs baseline: 2.7461x; 2.7461x over previous
import functools

import jax
import jax.numpy as jnp
from jax import lax
from jax.experimental import pallas as pl

N = 10000
E = 320000
N16 = N + 16
BN = 2000
GRID = N // BN


def _tc_call(body, out_shapes, in_specs, out_specs):
    return pl.pallas_call(
        body,
        grid=(GRID,),
        in_specs=in_specs,
        out_specs=out_specs,
        out_shape=out_shapes,
    )


def _row_spec(w):
    return pl.BlockSpec((BN, w), lambda i: (i, 0))


def _pair_spec(w):
    return pl.BlockSpec((2, BN, w), lambda i: (0, i, 0))


def _full_spec(r, c):
    return pl.BlockSpec((r, c), lambda i: (0, 0))


def _dis_from_counts(cb):
    deg = 1.0 + cb[0, :, 0] + cb[1, :, 0]
    return lax.rsqrt(deg)


def _b1_body(x_ref, c_ref, w1_ref, wsk_ref, bsk_ref, yw1_ref, skip02_ref):
    dis = _dis_from_counts(c_ref[...])
    xb = x_ref[...]
    xw = jnp.dot(xb, w1_ref[...], preferred_element_type=jnp.float32)
    yw1_ref[...] = dis[:, None] * xw
    skip02_ref[...] = jnp.dot(xb, wsk_ref[...], preferred_element_type=jnp.float32) + bsk_ref[...]


def _b2_body(s1_ref, yw1_ref, c_ref, w2_ref, b1_ref, wsk_ref, bsk_ref,
             yw2_ref, skip13_ref):
    dis = _dis_from_counts(c_ref[...])
    s = s1_ref[0] + s1_ref[1] + yw1_ref[...]
    x1 = jax.nn.relu(dis[:, None] * s + b1_ref[...])
    xw2 = jnp.dot(x1, w2_ref[...], preferred_element_type=jnp.float32)
    yw2_ref[...] = dis[:, None] * xw2
    skip13_ref[...] = jnp.dot(x1, wsk_ref[...], preferred_element_type=jnp.float32) + bsk_ref[...]


def _b3_body(s2_ref, yw2_ref, skip02_ref, c_ref, w3_ref, b2_ref, yw3_ref):
    dis = _dis_from_counts(c_ref[...])
    s = s2_ref[0] + s2_ref[1] + yw2_ref[...]
    x2 = jax.nn.relu(dis[:, None] * s + b2_ref[...] + skip02_ref[...])
    xw3 = jnp.dot(x2, w3_ref[...], preferred_element_type=jnp.float32)
    yw3_ref[...] = dis[:, None] * xw3


def _b4_body(s3_ref, yw3_ref, skip13_ref, c_ref, wl_ref, b3_ref, bl_ref, out_ref):
    dis = _dis_from_counts(c_ref[...])
    s = s3_ref[0] + s3_ref[1] + yw3_ref[...]
    x3 = jax.nn.relu(dis[:, None] * s + b3_ref[...] + skip13_ref[...])
    z = jnp.dot(x3, wl_ref[...], preferred_element_type=jnp.float32) + bl_ref[...]
    out_ref[...] = jax.nn.sigmoid(z)


def _counts_pair(dst):
    c = jnp.zeros((N,), jnp.float32).at[dst].add(1.0)
    pair = jnp.zeros((2, N16, 16), jnp.float32)
    return pair.at[0, :N, :].set(c[:, None])


def _agg_pair(yw, src, dst):
    w = yw.shape[1]
    s = jnp.zeros((N, w), jnp.float32).at[dst].add(yw[src])
    pair = jnp.zeros((2, N16, w), jnp.float32)
    return pair.at[0, :N, :].set(s)


def kernel(x, edge_index, W1, b1, W_skip02, b_skip02, W2, b2, W_skip13,
           b_skip13, W3, b3, W_lin, b_lin):
    src = edge_index[0]
    dst = edge_index[1]
    b1r = b1.reshape(1, -1)
    b2r = b2.reshape(1, -1)
    b3r = b3.reshape(1, -1)
    bsk02 = b_skip02.reshape(1, -1)
    bsk13 = b_skip13.reshape(1, -1)
    blr = b_lin.reshape(1, -1)

    counts = _counts_pair(dst)

    yw1, skip02 = _tc_call(
        _b1_body,
        (jax.ShapeDtypeStruct((N, 64), jnp.float32),
         jax.ShapeDtypeStruct((N, 32), jnp.float32)),
        [_row_spec(128), _pair_spec(16), _full_spec(128, 64),
         _full_spec(128, 32), _full_spec(1, 32)],
        [_row_spec(64), _row_spec(32)],
    )(x, counts, W1, W_skip02, bsk02)

    s1 = _agg_pair(yw1, src, dst)

    yw2, skip13 = _tc_call(
        _b2_body,
        (jax.ShapeDtypeStruct((N, 32), jnp.float32),
         jax.ShapeDtypeStruct((N, 16), jnp.float32)),
        [_pair_spec(64), _row_spec(64), _pair_spec(16), _full_spec(64, 32),
         _full_spec(1, 64), _full_spec(64, 16), _full_spec(1, 16)],
        [_row_spec(32), _row_spec(16)],
    )(s1, yw1, counts, W2, b1r, W_skip13, bsk13)

    s2 = _agg_pair(yw2, src, dst)

    yw3 = _tc_call(
        _b3_body,
        jax.ShapeDtypeStruct((N, 16), jnp.float32),
        [_pair_spec(32), _row_spec(32), _row_spec(32), _pair_spec(16),
         _full_spec(32, 16), _full_spec(1, 32)],
        _row_spec(16),
    )(s2, yw2, skip02, counts, W3, b2r)

    s3 = _agg_pair(yw3, src, dst)

    out = _tc_call(
        _b4_body,
        jax.ShapeDtypeStruct((N, 1), jnp.float32),
        [_pair_spec(16), _row_spec(16), _row_spec(16), _pair_spec(16),
         _full_spec(16, 1), _full_spec(1, 16), _full_spec(1, 1)],
        _row_spec(1),
    )(s3, yw3, skip13, counts, W_lin, b3r, blr)

    return out


# speedup vs baseline: 27.0634x; 9.8553x over previous
import functools

import jax
import jax.numpy as jnp
from jax import lax
from jax.experimental import pallas as pl
from jax.experimental.pallas import tpu as pltpu
from jax.experimental.pallas import tpu_sc as plsc

N = 10000
E = 320000
BN = 2000
GRID = N // BN

NC = 2
NS = 16
NW = NC * NS
CH = 128
PERW = E // NW
NCHUNK = 80
PERW_PAD = NCHUNK * CH
N16 = 10112
ROWS_PER_TILE = N16 // NS


def _tc_call(body, out_shapes, in_specs, out_specs):
    return pl.pallas_call(
        body,
        grid=(GRID,),
        in_specs=in_specs,
        out_specs=out_specs,
        out_shape=out_shapes,
    )


def _row_spec(w):
    return pl.BlockSpec((BN, w), lambda i: (i, 0))


def _pair_spec(w):
    return pl.BlockSpec((2, BN, w), lambda i: (0, i, 0))


def _full_spec(r, c):
    return pl.BlockSpec((r, c), lambda i: (0, 0))


def _dis_from_counts(cb):
    deg = 1.0 + cb[0, :, 0] + cb[1, :, 0]
    return lax.rsqrt(deg)


def _b1_body(x_ref, c_ref, w1_ref, wsk_ref, bsk_ref, yw1_ref, skip02_ref):
    dis = _dis_from_counts(c_ref[...])
    xb = x_ref[...]
    xw = jnp.dot(xb, w1_ref[...], preferred_element_type=jnp.float32)
    yw1_ref[...] = dis[:, None] * xw
    skip02_ref[...] = jnp.dot(xb, wsk_ref[...], preferred_element_type=jnp.float32) + bsk_ref[...]


def _b2_body(s1_ref, yw1_ref, c_ref, w2_ref, b1_ref, wsk_ref, bsk_ref,
             yw2_ref, skip13_ref):
    dis = _dis_from_counts(c_ref[...])
    s = s1_ref[0] + s1_ref[1] + yw1_ref[...]
    x1 = jax.nn.relu(dis[:, None] * s + b1_ref[...])
    xw2 = jnp.dot(x1, w2_ref[...], preferred_element_type=jnp.float32)
    yw2_ref[...] = dis[:, None] * xw2
    skip13_ref[...] = jnp.dot(x1, wsk_ref[...], preferred_element_type=jnp.float32) + bsk_ref[...]


def _b3_body(s2_ref, yw2_ref, skip02_ref, c_ref, w3_ref, b2_ref, yw3_ref):
    dis = _dis_from_counts(c_ref[...])
    s = s2_ref[0] + s2_ref[1] + yw2_ref[...]
    x2 = jax.nn.relu(dis[:, None] * s + b2_ref[...] + skip02_ref[...])
    xw3 = jnp.dot(x2, w3_ref[...], preferred_element_type=jnp.float32)
    yw3_ref[...] = dis[:, None] * xw3


def _b4_body(s3_ref, yw3_ref, skip13_ref, c_ref, wl_ref, b3_ref, bl_ref, out_ref):
    dis = _dis_from_counts(c_ref[...])
    s = s3_ref[0] + s3_ref[1] + yw3_ref[...]
    x3 = jax.nn.relu(dis[:, None] * s + b3_ref[...] + skip13_ref[...])
    z = jnp.dot(x3, wl_ref[...], preferred_element_type=jnp.float32) + bl_ref[...]
    out_ref[...] = jax.nn.sigmoid(z)


def _sc_mesh():
    return plsc.VectorSubcoreMesh(core_axis_name="c", subcore_axis_name="s",
                                  num_cores=NC, num_subcores=NS)


_SC_PARAMS = pltpu.CompilerParams(use_tc_tiling_on_sc=False)


def _pad_edges(idx, pad_vals):
    a2 = idx.reshape(NW, PERW)
    pad = jnp.broadcast_to(pad_vals, (NW, PERW_PAD - PERW))
    return jnp.concatenate([a2, pad], axis=1).reshape(NW * NCHUNK, CH)


@functools.partial(
    pl.kernel,
    out_type=jax.ShapeDtypeStruct((NC, N16, 16), jnp.float32),
    mesh=_sc_mesh(),
    compiler_params=_SC_PARAMS,
    scratch_types=[
        pltpu.VMEM((NCHUNK, CH), jnp.int32),
        pltpu.VMEM((CH, 16), jnp.float32),
        pltpu.VMEM_SHARED((N16, 16), jnp.float32),
    ],
)
def _deg_kernel(dst_hbm, ones_hbm, zeros_hbm, out_hbm, idx_v, ones_v, accum):
    c = lax.axis_index("c")
    s = lax.axis_index("s")
    wid = s * NC + c
    r0 = s * ROWS_PER_TILE
    pltpu.sync_copy(zeros_hbm.at[pl.ds(r0, ROWS_PER_TILE)],
                    accum.at[pl.ds(r0, ROWS_PER_TILE)])
    pltpu.sync_copy(dst_hbm.at[pl.ds(wid * NCHUNK, NCHUNK)], idx_v)
    pltpu.sync_copy(ones_hbm, ones_v)
    plsc.subcore_barrier()

    def body(j, carry):
        pltpu.sync_copy(ones_v, accum.at[idx_v.at[j]], add=True)
        return carry

    lax.fori_loop(0, NCHUNK, body, 0)
    plsc.subcore_barrier()
    pltpu.sync_copy(accum.at[pl.ds(r0, ROWS_PER_TILE)],
                    out_hbm.at[c].at[pl.ds(r0, ROWS_PER_TILE)])


def _make_agg(w):
    @functools.partial(
        pl.kernel,
        out_type=jax.ShapeDtypeStruct((NC, N16, w), jnp.float32),
        mesh=_sc_mesh(),
        compiler_params=_SC_PARAMS,
        scratch_types=[
            pltpu.VMEM((NCHUNK, CH), jnp.int32),
            pltpu.VMEM((NCHUNK, CH), jnp.int32),
            pltpu.VMEM((CH, w), jnp.float32),
            pltpu.VMEM_SHARED((N16, w), jnp.float32),
            pltpu.SemaphoreType.DMA,
        ],
    )
    def _agg(yw_hbm, src_hbm, dst_hbm, zeros_hbm, out_hbm,
             src_v, dst_v, rows_v, accum, sem):
        c = lax.axis_index("c")
        s = lax.axis_index("s")
        wid = s * NC + c
        r0 = s * ROWS_PER_TILE
        pltpu.sync_copy(zeros_hbm.at[pl.ds(r0, ROWS_PER_TILE)],
                        accum.at[pl.ds(r0, ROWS_PER_TILE)])
        pltpu.sync_copy(src_hbm.at[pl.ds(wid * NCHUNK, NCHUNK)], src_v)
        pltpu.sync_copy(dst_hbm.at[pl.ds(wid * NCHUNK, NCHUNK)], dst_v)
        plsc.subcore_barrier()

        def body(j, carry):
            pltpu.async_copy(yw_hbm.at[src_v.at[j]], rows_v, sem).wait()
            pltpu.sync_copy(rows_v, accum.at[dst_v.at[j]], add=True)
            return carry

        lax.fori_loop(0, NCHUNK, body, 0)
        plsc.subcore_barrier()
        pltpu.sync_copy(accum.at[pl.ds(r0, ROWS_PER_TILE)],
                        out_hbm.at[c].at[pl.ds(r0, ROWS_PER_TILE)])

    return _agg


_agg64 = _make_agg(64)
_agg32 = _make_agg(32)
_agg16 = _make_agg(16)
_AGG = {64: _agg64, 32: _agg32, 16: _agg16}


def _counts_pair(dstp, ones, zeros16):
    return _deg_kernel(dstp, ones, zeros16)


def _agg_pair(yw, srcp, dstp, zeros):
    return _AGG[yw.shape[1]](yw, srcp, dstp, zeros)


def kernel(x, edge_index, W1, b1, W_skip02, b_skip02, W2, b2, W_skip13,
           b_skip13, W3, b3, W_lin, b_lin):
    pad_src = jnp.arange(PERW_PAD - PERW, dtype=jnp.int32) % N
    pad_dst = N + (jnp.arange(PERW_PAD - PERW, dtype=jnp.int32) % 16)
    srcp = _pad_edges(edge_index[0], pad_src)
    dstp = _pad_edges(edge_index[1], pad_dst)
    ones = jnp.ones((CH, 16), jnp.float32)
    zeros16 = jnp.zeros((N16, 16), jnp.float32)
    zeros64 = jnp.zeros((N16, 64), jnp.float32)
    zeros32 = jnp.zeros((N16, 32), jnp.float32)
    b1r = b1.reshape(1, -1)
    b2r = b2.reshape(1, -1)
    b3r = b3.reshape(1, -1)
    bsk02 = b_skip02.reshape(1, -1)
    bsk13 = b_skip13.reshape(1, -1)
    blr = b_lin.reshape(1, -1)

    counts = _counts_pair(dstp, ones, zeros16)

    yw1, skip02 = _tc_call(
        _b1_body,
        (jax.ShapeDtypeStruct((N, 64), jnp.float32),
         jax.ShapeDtypeStruct((N, 32), jnp.float32)),
        [_row_spec(128), _pair_spec(16), _full_spec(128, 64),
         _full_spec(128, 32), _full_spec(1, 32)],
        [_row_spec(64), _row_spec(32)],
    )(x, counts, W1, W_skip02, bsk02)

    s1 = _agg_pair(yw1, srcp, dstp, zeros64)

    yw2, skip13 = _tc_call(
        _b2_body,
        (jax.ShapeDtypeStruct((N, 32), jnp.float32),
         jax.ShapeDtypeStruct((N, 16), jnp.float32)),
        [_pair_spec(64), _row_spec(64), _pair_spec(16), _full_spec(64, 32),
         _full_spec(1, 64), _full_spec(64, 16), _full_spec(1, 16)],
        [_row_spec(32), _row_spec(16)],
    )(s1, yw1, counts, W2, b1r, W_skip13, bsk13)

    s2 = _agg_pair(yw2, srcp, dstp, zeros32)

    yw3 = _tc_call(
        _b3_body,
        jax.ShapeDtypeStruct((N, 16), jnp.float32),
        [_pair_spec(32), _row_spec(32), _row_spec(32), _pair_spec(16),
         _full_spec(32, 16), _full_spec(1, 32)],
        _row_spec(16),
    )(s2, yw2, skip02, counts, W3, b2r)

    s3 = _agg_pair(yw3, srcp, dstp, zeros16)

    out = _tc_call(
        _b4_body,
        jax.ShapeDtypeStruct((N, 1), jnp.float32),
        [_pair_spec(16), _row_spec(16), _row_spec(16), _pair_spec(16),
         _full_spec(16, 1), _full_spec(1, 16), _full_spec(1, 1)],
        _row_spec(1),
    )(s3, yw3, skip13, counts, W_lin, b3r, blr)

    return out


# speedup vs baseline: 36.6714x; 1.3550x over previous
import functools

import jax
import jax.numpy as jnp
from jax import lax
from jax.experimental import pallas as pl
from jax.experimental.pallas import tpu as pltpu
from jax.experimental.pallas import tpu_sc as plsc

N = 10000
E = 320000
BN = 2000
GRID = N // BN

NC = 2
NS = 16
NW = NC * NS
CH = 128
PERW = E // NW
NCHUNK = 80
PERW_PAD = NCHUNK * CH
N16 = 10112
ROWS_PER_TILE = N16 // NS
KDEPTH = 8


def _tc_call(body, out_shapes, in_specs, out_specs):
    return pl.pallas_call(
        body,
        grid=(GRID,),
        in_specs=in_specs,
        out_specs=out_specs,
        out_shape=out_shapes,
    )


def _row_spec(w):
    return pl.BlockSpec((BN, w), lambda i: (i, 0))


def _pair_spec(w):
    return pl.BlockSpec((2, BN, w), lambda i: (0, i, 0))


def _full_spec(r, c):
    return pl.BlockSpec((r, c), lambda i: (0, 0))


def _dis_from_counts(cb):
    deg = 1.0 + cb[0, :, 0] + cb[1, :, 0]
    return lax.rsqrt(deg)


def _b1_body(x_ref, c_ref, w1_ref, wsk_ref, bsk_ref, yw1_ref, skip02_ref):
    dis = _dis_from_counts(c_ref[...])
    xb = x_ref[...]
    xw = jnp.dot(xb, w1_ref[...], preferred_element_type=jnp.float32)
    yw1_ref[...] = dis[:, None] * xw
    skip02_ref[...] = jnp.dot(xb, wsk_ref[...], preferred_element_type=jnp.float32) + bsk_ref[...]


def _b2_body(s1_ref, yw1_ref, c_ref, w2_ref, b1_ref, wsk_ref, bsk_ref,
             yw2_ref, skip13_ref):
    dis = _dis_from_counts(c_ref[...])
    s = s1_ref[0] + s1_ref[1] + yw1_ref[...]
    x1 = jax.nn.relu(dis[:, None] * s + b1_ref[...])
    xw2 = jnp.dot(x1, w2_ref[...], preferred_element_type=jnp.float32)
    yw2_ref[...] = dis[:, None] * xw2
    skip13_ref[...] = jnp.dot(x1, wsk_ref[...], preferred_element_type=jnp.float32) + bsk_ref[...]


def _b3_body(s2_ref, yw2_ref, skip02_ref, c_ref, w3_ref, b2_ref, yw3_ref):
    dis = _dis_from_counts(c_ref[...])
    s = s2_ref[0] + s2_ref[1] + yw2_ref[...]
    x2 = jax.nn.relu(dis[:, None] * s + b2_ref[...] + skip02_ref[...])
    xw3 = jnp.dot(x2, w3_ref[...], preferred_element_type=jnp.float32)
    yw3_ref[...] = dis[:, None] * xw3


def _b4_body(s3_ref, yw3_ref, skip13_ref, c_ref, wl_ref, b3_ref, bl_ref, out_ref):
    dis = _dis_from_counts(c_ref[...])
    s = s3_ref[0] + s3_ref[1] + yw3_ref[...]
    x3 = jax.nn.relu(dis[:, None] * s + b3_ref[...] + skip13_ref[...])
    z = jnp.dot(x3, wl_ref[...], preferred_element_type=jnp.float32) + bl_ref[...]
    out_ref[...] = jax.nn.sigmoid(z)


def _sc_mesh():
    return plsc.VectorSubcoreMesh(core_axis_name="c", subcore_axis_name="s",
                                  num_cores=NC, num_subcores=NS)


_SC_PARAMS = pltpu.CompilerParams(use_tc_tiling_on_sc=False)


def _pad_edges(idx, pad_vals):
    a2 = idx.reshape(NW, PERW)
    pad = jnp.broadcast_to(pad_vals, (NW, PERW_PAD - PERW))
    return jnp.concatenate([a2, pad], axis=1).reshape(NW * NCHUNK, CH)


@functools.partial(
    pl.kernel,
    out_type=jax.ShapeDtypeStruct((NC, N16, 16), jnp.float32),
    mesh=_sc_mesh(),
    compiler_params=_SC_PARAMS,
    scratch_types=[
        pltpu.VMEM((NCHUNK, CH), jnp.int32),
        pltpu.VMEM((CH, 16), jnp.float32),
        pltpu.VMEM_SHARED((N16, 16), jnp.float32),
    ],
)
def _deg_kernel(dst_hbm, ones_hbm, zeros_hbm, out_hbm, idx_v, ones_v, accum):
    c = lax.axis_index("c")
    s = lax.axis_index("s")
    wid = s * NC + c
    r0 = s * ROWS_PER_TILE
    pltpu.sync_copy(zeros_hbm.at[pl.ds(r0, ROWS_PER_TILE)],
                    accum.at[pl.ds(r0, ROWS_PER_TILE)])
    pltpu.sync_copy(dst_hbm.at[pl.ds(wid * NCHUNK, NCHUNK)], idx_v)
    pltpu.sync_copy(ones_hbm, ones_v)
    plsc.subcore_barrier()

    def body(j, carry):
        pltpu.sync_copy(ones_v, accum.at[idx_v.at[j]], add=True)
        return carry

    lax.fori_loop(0, NCHUNK, body, 0)
    plsc.subcore_barrier()
    pltpu.sync_copy(accum.at[pl.ds(r0, ROWS_PER_TILE)],
                    out_hbm.at[c].at[pl.ds(r0, ROWS_PER_TILE)])


def _make_agg(w):
    @functools.partial(
        pl.kernel,
        out_type=jax.ShapeDtypeStruct((NC, N16, w), jnp.float32),
        mesh=_sc_mesh(),
        compiler_params=_SC_PARAMS,
        scratch_types=[
            pltpu.VMEM((NCHUNK, CH), jnp.int32),
            pltpu.VMEM((NCHUNK, CH), jnp.int32),
            pltpu.VMEM((KDEPTH, CH, w), jnp.float32),
            pltpu.VMEM_SHARED((N16, w), jnp.float32),
            pltpu.SemaphoreType.DMA,
        ],
    )
    def _agg(yw_hbm, src_hbm, dst_hbm, zeros_hbm, out_hbm,
             src_v, dst_v, rows_v, accum, sem):
        c = lax.axis_index("c")
        s = lax.axis_index("s")
        wid = s * NC + c
        r0 = s * ROWS_PER_TILE
        pltpu.sync_copy(zeros_hbm.at[pl.ds(r0, ROWS_PER_TILE)],
                        accum.at[pl.ds(r0, ROWS_PER_TILE)])
        pltpu.sync_copy(src_hbm.at[pl.ds(wid * NCHUNK, NCHUNK)], src_v)
        pltpu.sync_copy(dst_hbm.at[pl.ds(wid * NCHUNK, NCHUNK)], dst_v)
        plsc.subcore_barrier()

        def body(g, carry):
            descs = []
            for b in range(KDEPTH):
                j = g * KDEPTH + b
                descs.append(
                    pltpu.async_copy(yw_hbm.at[src_v.at[j]], rows_v.at[b], sem))
            for d in descs:
                d.wait()
            for b in range(KDEPTH):
                j = g * KDEPTH + b
                pltpu.sync_copy(rows_v.at[b], accum.at[dst_v.at[j]], add=True)
            return carry

        lax.fori_loop(0, NCHUNK // KDEPTH, body, 0)
        plsc.subcore_barrier()
        pltpu.sync_copy(accum.at[pl.ds(r0, ROWS_PER_TILE)],
                        out_hbm.at[c].at[pl.ds(r0, ROWS_PER_TILE)])

    return _agg


_agg64 = _make_agg(64)
_agg32 = _make_agg(32)
_agg16 = _make_agg(16)
_AGG = {64: _agg64, 32: _agg32, 16: _agg16}


def _counts_pair(dstp, ones, zeros16):
    return _deg_kernel(dstp, ones, zeros16)


def _agg_pair(yw, srcp, dstp, zeros):
    return _AGG[yw.shape[1]](yw, srcp, dstp, zeros)


def kernel(x, edge_index, W1, b1, W_skip02, b_skip02, W2, b2, W_skip13,
           b_skip13, W3, b3, W_lin, b_lin):
    pad_src = jnp.arange(PERW_PAD - PERW, dtype=jnp.int32) % N
    pad_dst = N + (jnp.arange(PERW_PAD - PERW, dtype=jnp.int32) % 16)
    srcp = _pad_edges(edge_index[0], pad_src)
    dstp = _pad_edges(edge_index[1], pad_dst)
    ones = jnp.ones((CH, 16), jnp.float32)
    zeros16 = jnp.zeros((N16, 16), jnp.float32)
    zeros64 = jnp.zeros((N16, 64), jnp.float32)
    zeros32 = jnp.zeros((N16, 32), jnp.float32)
    b1r = b1.reshape(1, -1)
    b2r = b2.reshape(1, -1)
    b3r = b3.reshape(1, -1)
    bsk02 = b_skip02.reshape(1, -1)
    bsk13 = b_skip13.reshape(1, -1)
    blr = b_lin.reshape(1, -1)

    counts = _counts_pair(dstp, ones, zeros16)

    yw1, skip02 = _tc_call(
        _b1_body,
        (jax.ShapeDtypeStruct((N, 64), jnp.float32),
         jax.ShapeDtypeStruct((N, 32), jnp.float32)),
        [_row_spec(128), _pair_spec(16), _full_spec(128, 64),
         _full_spec(128, 32), _full_spec(1, 32)],
        [_row_spec(64), _row_spec(32)],
    )(x, counts, W1, W_skip02, bsk02)

    s1 = _agg_pair(yw1, srcp, dstp, zeros64)

    yw2, skip13 = _tc_call(
        _b2_body,
        (jax.ShapeDtypeStruct((N, 32), jnp.float32),
         jax.ShapeDtypeStruct((N, 16), jnp.float32)),
        [_pair_spec(64), _row_spec(64), _pair_spec(16), _full_spec(64, 32),
         _full_spec(1, 64), _full_spec(64, 16), _full_spec(1, 16)],
        [_row_spec(32), _row_spec(16)],
    )(s1, yw1, counts, W2, b1r, W_skip13, bsk13)

    s2 = _agg_pair(yw2, srcp, dstp, zeros32)

    yw3 = _tc_call(
        _b3_body,
        jax.ShapeDtypeStruct((N, 16), jnp.float32),
        [_pair_spec(32), _row_spec(32), _row_spec(32), _pair_spec(16),
         _full_spec(32, 16), _full_spec(1, 32)],
        _row_spec(16),
    )(s2, yw2, skip02, counts, W3, b2r)

    s3 = _agg_pair(yw3, srcp, dstp, zeros16)

    out = _tc_call(
        _b4_body,
        jax.ShapeDtypeStruct((N, 1), jnp.float32),
        [_pair_spec(16), _row_spec(16), _row_spec(16), _pair_spec(16),
         _full_spec(16, 1), _full_spec(1, 16), _full_spec(1, 1)],
        _row_spec(1),
    )(s3, yw3, skip13, counts, W_lin, b3r, blr)

    return out


# speedup vs baseline: 43.2712x; 1.1800x over previous
import functools

import jax
import jax.numpy as jnp
from jax import lax
from jax.experimental import pallas as pl
from jax.experimental.pallas import tpu as pltpu
from jax.experimental.pallas import tpu_sc as plsc

N = 10000
E = 320000
BN = 2000
GRID = N // BN

NC = 2
NS = 16
NW = NC * NS
CH = 128
PERW = E // NW
NCHUNK = 80
PERW_PAD = NCHUNK * CH
N16 = 10112
ROWS_PER_TILE = N16 // NS
KDEPTH = 4
NGROUP = NCHUNK // KDEPTH


def _tc_call(body, out_shapes, in_specs, out_specs):
    return pl.pallas_call(
        body,
        grid=(GRID,),
        in_specs=in_specs,
        out_specs=out_specs,
        out_shape=out_shapes,
    )


def _row_spec(w):
    return pl.BlockSpec((BN, w), lambda i: (i, 0))


def _pair_spec(w):
    return pl.BlockSpec((2, BN, w), lambda i: (0, i, 0))


def _full_spec(r, c):
    return pl.BlockSpec((r, c), lambda i: (0, 0))


def _dis_from_counts(cb):
    deg = 1.0 + cb[0, :, 0] + cb[1, :, 0]
    return lax.rsqrt(deg)


def _b1_body(x_ref, c_ref, w1_ref, wsk_ref, bsk_ref, yw1_ref, skip02_ref):
    dis = _dis_from_counts(c_ref[...])
    xb = x_ref[...]
    xw = jnp.dot(xb, w1_ref[...], preferred_element_type=jnp.float32)
    yw1_ref[...] = dis[:, None] * xw
    skip02_ref[...] = jnp.dot(xb, wsk_ref[...], preferred_element_type=jnp.float32) + bsk_ref[...]


def _b2_body(s1_ref, yw1_ref, c_ref, w2_ref, b1_ref, wsk_ref, bsk_ref,
             yw2_ref, skip13_ref):
    dis = _dis_from_counts(c_ref[...])
    s = s1_ref[0] + s1_ref[1] + yw1_ref[...]
    x1 = jax.nn.relu(dis[:, None] * s + b1_ref[...])
    xw2 = jnp.dot(x1, w2_ref[...], preferred_element_type=jnp.float32)
    yw2_ref[...] = dis[:, None] * xw2
    skip13_ref[...] = jnp.dot(x1, wsk_ref[...], preferred_element_type=jnp.float32) + bsk_ref[...]


def _b3_body(s2_ref, yw2_ref, skip02_ref, c_ref, w3_ref, b2_ref, yw3_ref):
    dis = _dis_from_counts(c_ref[...])
    s = s2_ref[0] + s2_ref[1] + yw2_ref[...]
    x2 = jax.nn.relu(dis[:, None] * s + b2_ref[...] + skip02_ref[...])
    xw3 = jnp.dot(x2, w3_ref[...], preferred_element_type=jnp.float32)
    yw3_ref[...] = dis[:, None] * xw3


def _b4_body(s3_ref, yw3_ref, skip13_ref, c_ref, wl_ref, b3_ref, bl_ref, out_ref):
    dis = _dis_from_counts(c_ref[...])
    s = s3_ref[0] + s3_ref[1] + yw3_ref[...]
    x3 = jax.nn.relu(dis[:, None] * s + b3_ref[...] + skip13_ref[...])
    z = jnp.dot(x3, wl_ref[...], preferred_element_type=jnp.float32) + bl_ref[...]
    out_ref[...] = jax.nn.sigmoid(z)


def _sc_mesh():
    return plsc.VectorSubcoreMesh(core_axis_name="c", subcore_axis_name="s",
                                  num_cores=NC, num_subcores=NS)


_SC_PARAMS = pltpu.CompilerParams(use_tc_tiling_on_sc=False)


def _pad_edges(idx, pad_vals):
    a2 = idx.reshape(NW, PERW)
    pad = jnp.broadcast_to(pad_vals, (NW, PERW_PAD - PERW))
    return jnp.concatenate([a2, pad], axis=1).reshape(NW * NCHUNK, CH)


@functools.partial(
    pl.kernel,
    out_type=jax.ShapeDtypeStruct((NC, N16, 16), jnp.float32),
    mesh=_sc_mesh(),
    compiler_params=_SC_PARAMS,
    scratch_types=[
        pltpu.VMEM((NCHUNK, CH), jnp.int32),
        pltpu.VMEM((CH, 16), jnp.float32),
        pltpu.VMEM_SHARED((N16, 16), jnp.float32),
    ],
)
def _deg_kernel(dst_hbm, ones_hbm, zeros_hbm, out_hbm, idx_v, ones_v, accum):
    c = lax.axis_index("c")
    s = lax.axis_index("s")
    wid = s * NC + c
    r0 = s * ROWS_PER_TILE
    pltpu.sync_copy(zeros_hbm.at[pl.ds(r0, ROWS_PER_TILE)],
                    accum.at[pl.ds(r0, ROWS_PER_TILE)])
    pltpu.sync_copy(dst_hbm.at[pl.ds(wid * NCHUNK, NCHUNK)], idx_v)
    pltpu.sync_copy(ones_hbm, ones_v)
    plsc.subcore_barrier()

    def body(j, carry):
        pltpu.sync_copy(ones_v, accum.at[idx_v.at[j]], add=True)
        return carry

    lax.fori_loop(0, NCHUNK, body, 0)
    plsc.subcore_barrier()
    pltpu.sync_copy(accum.at[pl.ds(r0, ROWS_PER_TILE)],
                    out_hbm.at[c].at[pl.ds(r0, ROWS_PER_TILE)])


def _make_agg(w):
    @functools.partial(
        pl.kernel,
        out_type=jax.ShapeDtypeStruct((NC, N16, w), jnp.float32),
        mesh=_sc_mesh(),
        compiler_params=_SC_PARAMS,
        scratch_types=[
            pltpu.VMEM((NCHUNK, CH), jnp.int32),
            pltpu.VMEM((NCHUNK, CH), jnp.int32),
            pltpu.VMEM((2, KDEPTH, CH, w), jnp.float32),
            pltpu.VMEM_SHARED((N16, w), jnp.float32),
            pltpu.SemaphoreType.DMA,
            pltpu.SemaphoreType.DMA,
        ],
    )
    def _agg(yw_hbm, src_hbm, dst_hbm, zeros_hbm, out_hbm,
             src_v, dst_v, rows_v, accum, gsem, ssem):
        c = lax.axis_index("c")
        s = lax.axis_index("s")
        wid = s * NC + c
        r0 = s * ROWS_PER_TILE
        pltpu.sync_copy(zeros_hbm.at[pl.ds(r0, ROWS_PER_TILE)],
                        accum.at[pl.ds(r0, ROWS_PER_TILE)])
        pltpu.sync_copy(src_hbm.at[pl.ds(wid * NCHUNK, NCHUNK)], src_v)
        pltpu.sync_copy(dst_hbm.at[pl.ds(wid * NCHUNK, NCHUNK)], dst_v)
        plsc.subcore_barrier()

        def fire_gathers(g, p):
            for b in range(KDEPTH):
                pltpu.async_copy(yw_hbm.at[src_v.at[g * KDEPTH + b]],
                                 rows_v.at[p, b], gsem)

        def fire_scatters(g, p):
            for b in range(KDEPTH):
                pltpu.async_copy(rows_v.at[p, b],
                                 accum.at[dst_v.at[g * KDEPTH + b]],
                                 ssem, add=True)

        def drain(sem, p):
            for b in range(KDEPTH):
                pltpu.make_async_copy(yw_hbm.at[pl.ds(0, CH)],
                                      rows_v.at[p, b], sem).wait()

        def step(g, cur, oth):
            drain(ssem, oth)
            fire_gathers(g + 1, oth)
            drain(gsem, cur)
            fire_scatters(g, cur)

        fire_gathers(0, 0)
        fire_gathers(1, 1)
        drain(gsem, 0)
        fire_scatters(0, 0)

        def body(i, carry):
            step(2 * i + 1, 1, 0)
            step(2 * i + 2, 0, 1)
            return carry

        lax.fori_loop(0, (NGROUP - 2) // 2, body, 0)
        drain(gsem, 1)
        fire_scatters(NGROUP - 1, 1)
        drain(ssem, 0)
        drain(ssem, 1)
        plsc.subcore_barrier()
        pltpu.sync_copy(accum.at[pl.ds(r0, ROWS_PER_TILE)],
                        out_hbm.at[c].at[pl.ds(r0, ROWS_PER_TILE)])

    return _agg


_agg64 = _make_agg(64)
_agg32 = _make_agg(32)
_agg16 = _make_agg(16)
_AGG = {64: _agg64, 32: _agg32, 16: _agg16}


def _counts_pair(dstp, ones, zeros16):
    return _deg_kernel(dstp, ones, zeros16)


def _agg_pair(yw, srcp, dstp, zeros):
    return _AGG[yw.shape[1]](yw, srcp, dstp, zeros)


def kernel(x, edge_index, W1, b1, W_skip02, b_skip02, W2, b2, W_skip13,
           b_skip13, W3, b3, W_lin, b_lin):
    pad_src = jnp.arange(PERW_PAD - PERW, dtype=jnp.int32) % N
    pad_dst = N + (jnp.arange(PERW_PAD - PERW, dtype=jnp.int32) % 16)
    srcp = _pad_edges(edge_index[0], pad_src)
    dstp = _pad_edges(edge_index[1], pad_dst)
    ones = jnp.ones((CH, 16), jnp.float32)
    zeros16 = jnp.zeros((N16, 16), jnp.float32)
    zeros64 = jnp.zeros((N16, 64), jnp.float32)
    zeros32 = jnp.zeros((N16, 32), jnp.float32)
    b1r = b1.reshape(1, -1)
    b2r = b2.reshape(1, -1)
    b3r = b3.reshape(1, -1)
    bsk02 = b_skip02.reshape(1, -1)
    bsk13 = b_skip13.reshape(1, -1)
    blr = b_lin.reshape(1, -1)

    counts = _counts_pair(dstp, ones, zeros16)

    yw1, skip02 = _tc_call(
        _b1_body,
        (jax.ShapeDtypeStruct((N, 64), jnp.float32),
         jax.ShapeDtypeStruct((N, 32), jnp.float32)),
        [_row_spec(128), _pair_spec(16), _full_spec(128, 64),
         _full_spec(128, 32), _full_spec(1, 32)],
        [_row_spec(64), _row_spec(32)],
    )(x, counts, W1, W_skip02, bsk02)

    s1 = _agg_pair(yw1, srcp, dstp, zeros64)

    yw2, skip13 = _tc_call(
        _b2_body,
        (jax.ShapeDtypeStruct((N, 32), jnp.float32),
         jax.ShapeDtypeStruct((N, 16), jnp.float32)),
        [_pair_spec(64), _row_spec(64), _pair_spec(16), _full_spec(64, 32),
         _full_spec(1, 64), _full_spec(64, 16), _full_spec(1, 16)],
        [_row_spec(32), _row_spec(16)],
    )(s1, yw1, counts, W2, b1r, W_skip13, bsk13)

    s2 = _agg_pair(yw2, srcp, dstp, zeros32)

    yw3 = _tc_call(
        _b3_body,
        jax.ShapeDtypeStruct((N, 16), jnp.float32),
        [_pair_spec(32), _row_spec(32), _row_spec(32), _pair_spec(16),
         _full_spec(32, 16), _full_spec(1, 32)],
        _row_spec(16),
    )(s2, yw2, skip02, counts, W3, b2r)

    s3 = _agg_pair(yw3, srcp, dstp, zeros16)

    out = _tc_call(
        _b4_body,
        jax.ShapeDtypeStruct((N, 1), jnp.float32),
        [_pair_spec(16), _row_spec(16), _row_spec(16), _pair_spec(16),
         _full_spec(16, 1), _full_spec(1, 16), _full_spec(1, 1)],
        _row_spec(1),
    )(s3, yw3, skip13, counts, W_lin, b3r, blr)

    return out


# speedup vs baseline: 44.6025x; 1.0308x over previous
import functools

import jax
import jax.numpy as jnp
from jax import lax
from jax.experimental import pallas as pl
from jax.experimental.pallas import tpu as pltpu
from jax.experimental.pallas import tpu_sc as plsc

N = 10000
E = 320000
BN = 2000
GRID = N // BN

NC = 2
NS = 16
NW = NC * NS
CH = 128
PERW = E // NW
NCHUNK = 80
PERW_PAD = NCHUNK * CH
N16 = 10112
ROWS_PER_TILE = N16 // NS
KDEPTH = 4
NGROUP = NCHUNK // KDEPTH


def _tc_call(body, out_shapes, in_specs, out_specs):
    return pl.pallas_call(
        body,
        grid=(GRID,),
        in_specs=in_specs,
        out_specs=out_specs,
        out_shape=out_shapes,
    )


def _row_spec(w):
    return pl.BlockSpec((BN, w), lambda i: (i, 0))


def _pair_spec(w):
    return pl.BlockSpec((2, BN, w), lambda i: (0, i, 0))


def _full_spec(r, c):
    return pl.BlockSpec((r, c), lambda i: (0, 0))


def _vec_spec():
    return pl.BlockSpec((1, 1, BN), lambda i: (i, 0, 0))


def _dis_block(dis_ref):
    return dis_ref[0, 0, :]


def _dis_from_counts(cb):
    deg = 1.0 + cb[0, :, 0] + cb[1, :, 0]
    return lax.rsqrt(deg)


def _b0_body(x_ref, w1_ref, wsk_ref, bsk_ref, xw1_ref, skip02_ref):
    xb = x_ref[...]
    xw1_ref[...] = jnp.dot(xb, w1_ref[...], preferred_element_type=jnp.float32)
    skip02_ref[...] = jnp.dot(xb, wsk_ref[...], preferred_element_type=jnp.float32) + bsk_ref[...]


def _b1_body(xw1_ref, c_ref, yw1_ref, dis_ref):
    dis = _dis_from_counts(c_ref[...])
    yw1_ref[...] = dis[:, None] * xw1_ref[...]
    dis_ref[0, 0, :] = dis


def _b2_body(s1_ref, yw1_ref, dis_ref, w2_ref, b1_ref, wsk_ref, bsk_ref,
             yw2_ref, skip13_ref):
    dis = _dis_block(dis_ref)
    s = s1_ref[0] + s1_ref[1] + yw1_ref[...]
    x1 = jax.nn.relu(dis[:, None] * s + b1_ref[...])
    xw2 = jnp.dot(x1, w2_ref[...], preferred_element_type=jnp.float32)
    yw2_ref[...] = dis[:, None] * xw2
    skip13_ref[...] = jnp.dot(x1, wsk_ref[...], preferred_element_type=jnp.float32) + bsk_ref[...]


def _b3_body(s2_ref, yw2_ref, skip02_ref, dis_ref, w3_ref, b2_ref, yw3_ref):
    dis = _dis_block(dis_ref)
    s = s2_ref[0] + s2_ref[1] + yw2_ref[...]
    x2 = jax.nn.relu(dis[:, None] * s + b2_ref[...] + skip02_ref[...])
    xw3 = jnp.dot(x2, w3_ref[...], preferred_element_type=jnp.float32)
    yw3_ref[...] = dis[:, None] * xw3


def _b4_body(s3_ref, yw3_ref, skip13_ref, dis_ref, wl_ref, b3_ref, bl_ref, out_ref):
    dis = _dis_block(dis_ref)
    s = s3_ref[0] + s3_ref[1] + yw3_ref[...]
    x3 = jax.nn.relu(dis[:, None] * s + b3_ref[...] + skip13_ref[...])
    z = jnp.dot(x3, wl_ref[...], preferred_element_type=jnp.float32) + bl_ref[...]
    out_ref[...] = jax.nn.sigmoid(z)


def _sc_mesh():
    return plsc.VectorSubcoreMesh(core_axis_name="c", subcore_axis_name="s",
                                  num_cores=NC, num_subcores=NS)


_SC_PARAMS = pltpu.CompilerParams(use_tc_tiling_on_sc=False)


def _pad_edges(idx, pad_vals):
    a2 = idx.reshape(NW, PERW)
    pad = jnp.broadcast_to(pad_vals, (NW, PERW_PAD - PERW))
    return jnp.concatenate([a2, pad], axis=1).reshape(NW * NCHUNK, CH)


@functools.partial(
    pl.kernel,
    out_type=jax.ShapeDtypeStruct((NC, N16, 16), jnp.float32),
    mesh=_sc_mesh(),
    compiler_params=_SC_PARAMS,
    scratch_types=[
        pltpu.VMEM((NCHUNK, CH), jnp.int32),
        pltpu.VMEM((CH, 16), jnp.float32),
        pltpu.VMEM_SHARED((N16, 16), jnp.float32),
    ],
)
def _deg_kernel(dst_hbm, ones_hbm, zeros_hbm, out_hbm, idx_v, ones_v, accum):
    c = lax.axis_index("c")
    s = lax.axis_index("s")
    wid = s * NC + c
    r0 = s * ROWS_PER_TILE
    pltpu.sync_copy(zeros_hbm.at[pl.ds(r0, ROWS_PER_TILE)],
                    accum.at[pl.ds(r0, ROWS_PER_TILE)])
    pltpu.sync_copy(dst_hbm.at[pl.ds(wid * NCHUNK, NCHUNK)], idx_v)
    pltpu.sync_copy(ones_hbm, ones_v)
    plsc.subcore_barrier()

    def body(j, carry):
        pltpu.sync_copy(ones_v, accum.at[idx_v.at[j]], add=True)
        return carry

    lax.fori_loop(0, NCHUNK, body, 0)
    plsc.subcore_barrier()
    pltpu.sync_copy(accum.at[pl.ds(r0, ROWS_PER_TILE)],
                    out_hbm.at[c].at[pl.ds(r0, ROWS_PER_TILE)])


def _make_agg(w):
    @functools.partial(
        pl.kernel,
        out_type=jax.ShapeDtypeStruct((NC, N16, w), jnp.float32),
        mesh=_sc_mesh(),
        compiler_params=_SC_PARAMS,
        scratch_types=[
            pltpu.VMEM((NCHUNK, CH), jnp.int32),
            pltpu.VMEM((NCHUNK, CH), jnp.int32),
            pltpu.VMEM((2, KDEPTH, CH, w), jnp.float32),
            pltpu.VMEM_SHARED((N16, w), jnp.float32),
            pltpu.SemaphoreType.DMA,
            pltpu.SemaphoreType.DMA,
        ],
    )
    def _agg(yw_hbm, src_hbm, dst_hbm, zeros_hbm, out_hbm,
             src_v, dst_v, rows_v, accum, gsem, ssem):
        c = lax.axis_index("c")
        s = lax.axis_index("s")
        wid = s * NC + c
        r0 = s * ROWS_PER_TILE
        pltpu.sync_copy(zeros_hbm.at[pl.ds(r0, ROWS_PER_TILE)],
                        accum.at[pl.ds(r0, ROWS_PER_TILE)])
        pltpu.sync_copy(src_hbm.at[pl.ds(wid * NCHUNK, NCHUNK)], src_v)
        pltpu.sync_copy(dst_hbm.at[pl.ds(wid * NCHUNK, NCHUNK)], dst_v)
        plsc.subcore_barrier()

        def fire_gathers(g, p):
            for b in range(KDEPTH):
                pltpu.async_copy(yw_hbm.at[src_v.at[g * KDEPTH + b]],
                                 rows_v.at[p, b], gsem)

        def fire_scatters(g, p):
            for b in range(KDEPTH):
                pltpu.async_copy(rows_v.at[p, b],
                                 accum.at[dst_v.at[g * KDEPTH + b]],
                                 ssem, add=True)

        def drain(sem, p):
            for b in range(KDEPTH):
                pltpu.make_async_copy(yw_hbm.at[pl.ds(0, CH)],
                                      rows_v.at[p, b], sem).wait()

        def step(g, cur, oth):
            drain(ssem, oth)
            fire_gathers(g + 1, oth)
            drain(gsem, cur)
            fire_scatters(g, cur)

        fire_gathers(0, 0)
        fire_gathers(1, 1)
        drain(gsem, 0)
        fire_scatters(0, 0)

        def body(i, carry):
            step(2 * i + 1, 1, 0)
            step(2 * i + 2, 0, 1)
            return carry

        lax.fori_loop(0, (NGROUP - 2) // 2, body, 0)
        drain(gsem, 1)
        fire_scatters(NGROUP - 1, 1)
        drain(ssem, 0)
        drain(ssem, 1)
        plsc.subcore_barrier()
        pltpu.sync_copy(accum.at[pl.ds(r0, ROWS_PER_TILE)],
                        out_hbm.at[c].at[pl.ds(r0, ROWS_PER_TILE)])

    return _agg


_agg64 = _make_agg(64)
_agg32 = _make_agg(32)
_agg16 = _make_agg(16)
_AGG = {64: _agg64, 32: _agg32, 16: _agg16}


def _counts_pair(dstp, ones, zeros16):
    return _deg_kernel(dstp, ones, zeros16)


def _agg_pair(yw, srcp, dstp, zeros):
    return _AGG[yw.shape[1]](yw, srcp, dstp, zeros)


def kernel(x, edge_index, W1, b1, W_skip02, b_skip02, W2, b2, W_skip13,
           b_skip13, W3, b3, W_lin, b_lin):
    pad_src = jnp.arange(PERW_PAD - PERW, dtype=jnp.int32) % N
    pad_dst = N + (jnp.arange(PERW_PAD - PERW, dtype=jnp.int32) % 16)
    srcp = _pad_edges(edge_index[0], pad_src)
    dstp = _pad_edges(edge_index[1], pad_dst)
    ones = jnp.ones((CH, 16), jnp.float32)
    zeros16 = jnp.zeros((N16, 16), jnp.float32)
    zeros64 = jnp.zeros((N16, 64), jnp.float32)
    zeros32 = jnp.zeros((N16, 32), jnp.float32)
    b1r = b1.reshape(1, -1)
    b2r = b2.reshape(1, -1)
    b3r = b3.reshape(1, -1)
    bsk02 = b_skip02.reshape(1, -1)
    bsk13 = b_skip13.reshape(1, -1)
    blr = b_lin.reshape(1, -1)

    counts = _counts_pair(dstp, ones, zeros16)

    xw1, skip02 = _tc_call(
        _b0_body,
        (jax.ShapeDtypeStruct((N, 64), jnp.float32),
         jax.ShapeDtypeStruct((N, 32), jnp.float32)),
        [_row_spec(128), _full_spec(128, 64),
         _full_spec(128, 32), _full_spec(1, 32)],
        [_row_spec(64), _row_spec(32)],
    )(x, W1, W_skip02, bsk02)

    yw1, dis = _tc_call(
        _b1_body,
        (jax.ShapeDtypeStruct((N, 64), jnp.float32),
         jax.ShapeDtypeStruct((GRID, 1, BN), jnp.float32)),
        [_row_spec(64), _pair_spec(16)],
        [_row_spec(64), _vec_spec()],
    )(xw1, counts)

    s1 = _agg_pair(yw1, srcp, dstp, zeros64)

    yw2, skip13 = _tc_call(
        _b2_body,
        (jax.ShapeDtypeStruct((N, 32), jnp.float32),
         jax.ShapeDtypeStruct((N, 16), jnp.float32)),
        [_pair_spec(64), _row_spec(64), _vec_spec(), _full_spec(64, 32),
         _full_spec(1, 64), _full_spec(64, 16), _full_spec(1, 16)],
        [_row_spec(32), _row_spec(16)],
    )(s1, yw1, dis, W2, b1r, W_skip13, bsk13)

    s2 = _agg_pair(yw2, srcp, dstp, zeros32)

    yw3 = _tc_call(
        _b3_body,
        jax.ShapeDtypeStruct((N, 16), jnp.float32),
        [_pair_spec(32), _row_spec(32), _row_spec(32), _vec_spec(),
         _full_spec(32, 16), _full_spec(1, 32)],
        _row_spec(16),
    )(s2, yw2, skip02, dis, W3, b2r)

    s3 = _agg_pair(yw3, srcp, dstp, zeros16)

    out = _tc_call(
        _b4_body,
        jax.ShapeDtypeStruct((N, 1), jnp.float32),
        [_pair_spec(16), _row_spec(16), _row_spec(16), _vec_spec(),
         _full_spec(16, 1), _full_spec(1, 16), _full_spec(1, 1)],
        _row_spec(1),
    )(s3, yw3, skip13, dis, W_lin, b3r, blr)

    return out


# speedup vs baseline: 44.6337x; 1.0007x over previous
import functools

import jax
import jax.numpy as jnp
from jax import lax
from jax.experimental import pallas as pl
from jax.experimental.pallas import tpu as pltpu
from jax.experimental.pallas import tpu_sc as plsc

N = 10000
E = 320000
BN = 2000
GRID = N // BN

NC = 2
NS = 16
NW = NC * NS
CH = 128
PERW = E // NW
NCHUNK = 80
PERW_PAD = NCHUNK * CH
N16 = 10112
ROWS_PER_TILE = N16 // NS


def _tc_call(body, out_shapes, in_specs, out_specs):
    return pl.pallas_call(
        body,
        grid=(GRID,),
        in_specs=in_specs,
        out_specs=out_specs,
        out_shape=out_shapes,
    )


def _row_spec(w):
    return pl.BlockSpec((BN, w), lambda i: (i, 0))


def _pair_spec(w):
    return pl.BlockSpec((2, BN, w), lambda i: (0, i, 0))


def _full_spec(r, c):
    return pl.BlockSpec((r, c), lambda i: (0, 0))


def _vec_spec():
    return pl.BlockSpec((1, 1, BN), lambda i: (i, 0, 0))


def _dis_block(dis_ref):
    return dis_ref[0, 0, :]


def _dis_from_counts(cb):
    deg = 1.0 + cb[0, :, 0] + cb[1, :, 0]
    return lax.rsqrt(deg)


def _b0_body(x_ref, w1_ref, wsk_ref, bsk_ref, xw1_ref, skip02_ref):
    xb = x_ref[...]
    xw1_ref[...] = jnp.dot(xb, w1_ref[...], preferred_element_type=jnp.float32)
    skip02_ref[...] = jnp.dot(xb, wsk_ref[...], preferred_element_type=jnp.float32) + bsk_ref[...]


def _b1_body(xw1_ref, c_ref, yw1_ref, dis_ref):
    dis = _dis_from_counts(c_ref[...])
    yw1_ref[...] = dis[:, None] * xw1_ref[...]
    dis_ref[0, 0, :] = dis


def _b2_body(s1_ref, yw1_ref, dis_ref, w2_ref, b1_ref, wsk_ref, bsk_ref,
             yw2_ref, skip13_ref):
    dis = _dis_block(dis_ref)
    s = s1_ref[0] + s1_ref[1] + yw1_ref[...]
    x1 = jax.nn.relu(dis[:, None] * s + b1_ref[...])
    xw2 = jnp.dot(x1, w2_ref[...], preferred_element_type=jnp.float32)
    yw2_ref[...] = dis[:, None] * xw2
    skip13_ref[...] = jnp.dot(x1, wsk_ref[...], preferred_element_type=jnp.float32) + bsk_ref[...]


def _b3_body(s2_ref, yw2_ref, skip02_ref, dis_ref, w3_ref, b2_ref, yw3_ref):
    dis = _dis_block(dis_ref)
    s = s2_ref[0] + s2_ref[1] + yw2_ref[...]
    x2 = jax.nn.relu(dis[:, None] * s + b2_ref[...] + skip02_ref[...])
    xw3 = jnp.dot(x2, w3_ref[...], preferred_element_type=jnp.float32)
    yw3_ref[...] = dis[:, None] * xw3


def _b4_body(s3_ref, yw3_ref, skip13_ref, dis_ref, wl_ref, b3_ref, bl_ref, out_ref):
    dis = _dis_block(dis_ref)
    s = s3_ref[0] + s3_ref[1] + yw3_ref[...]
    x3 = jax.nn.relu(dis[:, None] * s + b3_ref[...] + skip13_ref[...])
    z = jnp.dot(x3, wl_ref[...], preferred_element_type=jnp.float32) + bl_ref[...]
    out_ref[...] = jax.nn.sigmoid(z)


def _sc_mesh():
    return plsc.VectorSubcoreMesh(core_axis_name="c", subcore_axis_name="s",
                                  num_cores=NC, num_subcores=NS)


_SC_PARAMS = pltpu.CompilerParams(use_tc_tiling_on_sc=False)


def _pad_edges(idx, pad_vals):
    a2 = idx.reshape(NW, PERW)
    pad = jnp.broadcast_to(pad_vals, (NW, PERW_PAD - PERW))
    return jnp.concatenate([a2, pad], axis=1).reshape(NW * NCHUNK, CH)


@functools.partial(
    pl.kernel,
    out_type=jax.ShapeDtypeStruct((NC, N16, 16), jnp.float32),
    mesh=_sc_mesh(),
    compiler_params=_SC_PARAMS,
    scratch_types=[
        pltpu.VMEM((NCHUNK, CH), jnp.int32),
        pltpu.VMEM((CH, 16), jnp.float32),
        pltpu.VMEM_SHARED((N16, 16), jnp.float32),
        pltpu.SemaphoreType.DMA,
    ],
)
def _deg_kernel(dst_hbm, ones_hbm, zeros_hbm, out_hbm, idx_v, ones_v, accum,
                ssem):
    c = lax.axis_index("c")
    s = lax.axis_index("s")
    wid = s * NC + c
    r0 = s * ROWS_PER_TILE
    pltpu.sync_copy(zeros_hbm, accum.at[pl.ds(r0, ROWS_PER_TILE)])
    pltpu.sync_copy(dst_hbm.at[pl.ds(wid * NCHUNK, NCHUNK)], idx_v)
    pltpu.sync_copy(ones_hbm, ones_v)
    plsc.subcore_barrier()

    def body(g, carry):
        for b in range(8):
            pltpu.async_copy(ones_v, accum.at[idx_v.at[g * 8 + b]], ssem,
                             add=True)
        return carry

    lax.fori_loop(0, NCHUNK // 8, body, 0)
    for _ in range(NCHUNK):
        pltpu.make_async_copy(ones_hbm, ones_v, ssem).wait()
    plsc.subcore_barrier()
    pltpu.sync_copy(accum.at[pl.ds(r0, ROWS_PER_TILE)],
                    out_hbm.at[c].at[pl.ds(r0, ROWS_PER_TILE)])


def _make_agg(w):
    kd = 4 if w == 64 else 8
    ngroup = NCHUNK // kd

    @functools.partial(
        pl.kernel,
        out_type=jax.ShapeDtypeStruct((NC, N16, w), jnp.float32),
        mesh=_sc_mesh(),
        compiler_params=_SC_PARAMS,
        scratch_types=[
            pltpu.VMEM((NCHUNK, CH), jnp.int32),
            pltpu.VMEM((NCHUNK, CH), jnp.int32),
            pltpu.VMEM((2, kd, CH, w), jnp.float32),
            pltpu.VMEM_SHARED((N16, w), jnp.float32),
            pltpu.SemaphoreType.DMA,
            pltpu.SemaphoreType.DMA,
        ],
    )
    def _agg(yw_hbm, src_hbm, dst_hbm, zeros_hbm, out_hbm,
             src_v, dst_v, rows_v, accum, gsem, ssem):
        c = lax.axis_index("c")
        s = lax.axis_index("s")
        wid = s * NC + c
        r0 = s * ROWS_PER_TILE
        pltpu.sync_copy(zeros_hbm, accum.at[pl.ds(r0, ROWS_PER_TILE)])
        pltpu.sync_copy(src_hbm.at[pl.ds(wid * NCHUNK, NCHUNK)], src_v)
        pltpu.sync_copy(dst_hbm.at[pl.ds(wid * NCHUNK, NCHUNK)], dst_v)
        plsc.subcore_barrier()

        def fire_gathers(g, p):
            for b in range(kd):
                pltpu.async_copy(yw_hbm.at[src_v.at[g * kd + b]],
                                 rows_v.at[p, b], gsem)

        def fire_scatters(g, p):
            for b in range(kd):
                pltpu.async_copy(rows_v.at[p, b],
                                 accum.at[dst_v.at[g * kd + b]],
                                 ssem, add=True)

        def drain(sem, p):
            for b in range(kd):
                pltpu.make_async_copy(yw_hbm.at[pl.ds(0, CH)],
                                      rows_v.at[p, b], sem).wait()

        def step(g, cur, oth):
            drain(ssem, oth)
            fire_gathers(g + 1, oth)
            drain(gsem, cur)
            fire_scatters(g, cur)

        fire_gathers(0, 0)
        fire_gathers(1, 1)
        drain(gsem, 0)
        fire_scatters(0, 0)

        def body(i, carry):
            step(2 * i + 1, 1, 0)
            step(2 * i + 2, 0, 1)
            return carry

        lax.fori_loop(0, (ngroup - 2) // 2, body, 0)
        drain(gsem, 1)
        fire_scatters(ngroup - 1, 1)
        drain(ssem, 0)
        drain(ssem, 1)
        plsc.subcore_barrier()
        pltpu.sync_copy(accum.at[pl.ds(r0, ROWS_PER_TILE)],
                        out_hbm.at[c].at[pl.ds(r0, ROWS_PER_TILE)])

    return _agg


_agg64 = _make_agg(64)
_agg32 = _make_agg(32)
_agg16 = _make_agg(16)
_AGG = {64: _agg64, 32: _agg32, 16: _agg16}


def _counts_pair(dstp, ones, zeros16):
    return _deg_kernel(dstp, ones, zeros16)


def _agg_pair(yw, srcp, dstp, zeros):
    return _AGG[yw.shape[1]](yw, srcp, dstp, zeros)


def kernel(x, edge_index, W1, b1, W_skip02, b_skip02, W2, b2, W_skip13,
           b_skip13, W3, b3, W_lin, b_lin):
    pad_src = jnp.arange(PERW_PAD - PERW, dtype=jnp.int32) % N
    pad_dst = N + (jnp.arange(PERW_PAD - PERW, dtype=jnp.int32) % 16)
    srcp = _pad_edges(edge_index[0], pad_src)
    dstp = _pad_edges(edge_index[1], pad_dst)
    ones = jnp.ones((CH, 16), jnp.float32)
    zeros16 = jnp.zeros((ROWS_PER_TILE, 16), jnp.float32)
    zeros64 = jnp.zeros((ROWS_PER_TILE, 64), jnp.float32)
    zeros32 = jnp.zeros((ROWS_PER_TILE, 32), jnp.float32)
    b1r = b1.reshape(1, -1)
    b2r = b2.reshape(1, -1)
    b3r = b3.reshape(1, -1)
    bsk02 = b_skip02.reshape(1, -1)
    bsk13 = b_skip13.reshape(1, -1)
    blr = b_lin.reshape(1, -1)

    counts = _counts_pair(dstp, ones, zeros16)

    xw1, skip02 = _tc_call(
        _b0_body,
        (jax.ShapeDtypeStruct((N, 64), jnp.float32),
         jax.ShapeDtypeStruct((N, 32), jnp.float32)),
        [_row_spec(128), _full_spec(128, 64),
         _full_spec(128, 32), _full_spec(1, 32)],
        [_row_spec(64), _row_spec(32)],
    )(x, W1, W_skip02, bsk02)

    yw1, dis = _tc_call(
        _b1_body,
        (jax.ShapeDtypeStruct((N, 64), jnp.float32),
         jax.ShapeDtypeStruct((GRID, 1, BN), jnp.float32)),
        [_row_spec(64), _pair_spec(16)],
        [_row_spec(64), _vec_spec()],
    )(xw1, counts)

    s1 = _agg_pair(yw1, srcp, dstp, zeros64)

    yw2, skip13 = _tc_call(
        _b2_body,
        (jax.ShapeDtypeStruct((N, 32), jnp.float32),
         jax.ShapeDtypeStruct((N, 16), jnp.float32)),
        [_pair_spec(64), _row_spec(64), _vec_spec(), _full_spec(64, 32),
         _full_spec(1, 64), _full_spec(64, 16), _full_spec(1, 16)],
        [_row_spec(32), _row_spec(16)],
    )(s1, yw1, dis, W2, b1r, W_skip13, bsk13)

    s2 = _agg_pair(yw2, srcp, dstp, zeros32)

    yw3 = _tc_call(
        _b3_body,
        jax.ShapeDtypeStruct((N, 16), jnp.float32),
        [_pair_spec(32), _row_spec(32), _row_spec(32), _vec_spec(),
         _full_spec(32, 16), _full_spec(1, 32)],
        _row_spec(16),
    )(s2, yw2, skip02, dis, W3, b2r)

    s3 = _agg_pair(yw3, srcp, dstp, zeros16)

    out = _tc_call(
        _b4_body,
        jax.ShapeDtypeStruct((N, 1), jnp.float32),
        [_pair_spec(16), _row_spec(16), _row_spec(16), _vec_spec(),
         _full_spec(16, 1), _full_spec(1, 16), _full_spec(1, 1)],
        _row_spec(1),
    )(s3, yw3, skip13, dis, W_lin, b3r, blr)

    return out


# speedup vs baseline: 45.1419x; 1.0114x over previous
import functools

import jax
import jax.numpy as jnp
from jax import lax
from jax.experimental import pallas as pl
from jax.experimental.pallas import tpu as pltpu
from jax.experimental.pallas import tpu_sc as plsc

N = 10000
E = 320000
BN = 2000
GRID = N // BN

NC = 2
NS = 16
NW = NC * NS
CH = 128
PERW = E // NW
NCHUNK = 80
PERW_PAD = NCHUNK * CH
N16 = 10112
ROWS_PER_TILE = N16 // NS


def _tc_call(body, out_shapes, in_specs, out_specs):
    return pl.pallas_call(
        body,
        grid=(GRID,),
        in_specs=in_specs,
        out_specs=out_specs,
        out_shape=out_shapes,
    )


def _row_spec(w):
    return pl.BlockSpec((BN, w), lambda i: (i, 0))


def _pair_spec(w):
    return pl.BlockSpec((2, BN, w), lambda i: (0, i, 0))


def _full_spec(r, c):
    return pl.BlockSpec((r, c), lambda i: (0, 0))


def _vec_spec():
    return pl.BlockSpec((1, 1, BN), lambda i: (i, 0, 0))


def _dis_block(dis_ref):
    return dis_ref[0, 0, :]


def _dis_from_counts(cb):
    deg = 1.0 + cb[0, :, 0] + cb[1, :, 0]
    return lax.rsqrt(deg)


def _b0_body(x_ref, w1_ref, wsk_ref, bsk_ref, xw1_ref, skip02_ref):
    xb = x_ref[...]
    xw1_ref[...] = jnp.dot(xb, w1_ref[...], preferred_element_type=jnp.float32)
    skip02_ref[...] = jnp.dot(xb, wsk_ref[...], preferred_element_type=jnp.float32) + bsk_ref[...]


def _b1_body(xw1_ref, c_ref, yw1_ref, dis_ref):
    dis = _dis_from_counts(c_ref[...])
    yw1_ref[...] = dis[:, None] * xw1_ref[...]
    dis_ref[0, 0, :] = dis


def _b2_body(s1_ref, yw1_ref, dis_ref, w2_ref, b1_ref, wsk_ref, bsk_ref,
             yw2_ref, skip13_ref):
    dis = _dis_block(dis_ref)
    s = s1_ref[0] + s1_ref[1] + yw1_ref[...]
    x1 = jax.nn.relu(dis[:, None] * s + b1_ref[...])
    xw2 = jnp.dot(x1, w2_ref[...], preferred_element_type=jnp.float32)
    yw2_ref[...] = dis[:, None] * xw2
    skip13_ref[...] = jnp.dot(x1, wsk_ref[...], preferred_element_type=jnp.float32) + bsk_ref[...]


def _b3_body(s2_ref, yw2_ref, skip02_ref, dis_ref, w3_ref, b2_ref, yw3_ref):
    dis = _dis_block(dis_ref)
    s = s2_ref[0] + s2_ref[1] + yw2_ref[...]
    x2 = jax.nn.relu(dis[:, None] * s + b2_ref[...] + skip02_ref[...])
    xw3 = jnp.dot(x2, w3_ref[...], preferred_element_type=jnp.float32)
    yw3_ref[...] = dis[:, None] * xw3


def _b4_body(s3_ref, yw3_ref, skip13_ref, dis_ref, wl_ref, b3_ref, bl_ref, out_ref):
    dis = _dis_block(dis_ref)
    s = s3_ref[0] + s3_ref[1] + yw3_ref[...]
    x3 = jax.nn.relu(dis[:, None] * s + b3_ref[...] + skip13_ref[...])
    z = jnp.dot(x3, wl_ref[...], preferred_element_type=jnp.float32) + bl_ref[...]
    out_ref[...] = jax.nn.sigmoid(z)


def _sc_mesh():
    return plsc.VectorSubcoreMesh(core_axis_name="c", subcore_axis_name="s",
                                  num_cores=NC, num_subcores=NS)


_SC_PARAMS = pltpu.CompilerParams(use_tc_tiling_on_sc=False)


def _pad_edges(idx, pad_vals):
    return jnp.concatenate([idx, pad_vals]).reshape(NW * NCHUNK, CH)


@functools.partial(
    pl.kernel,
    out_type=jax.ShapeDtypeStruct((NC, N16, 16), jnp.float32),
    mesh=_sc_mesh(),
    compiler_params=_SC_PARAMS,
    scratch_types=[
        pltpu.VMEM((NCHUNK, CH), jnp.int32),
        pltpu.VMEM((CH, 16), jnp.float32),
        pltpu.VMEM_SHARED((N16, 16), jnp.float32),
        pltpu.SemaphoreType.DMA,
    ],
)
def _deg_kernel(dst_hbm, ones_hbm, zeros_hbm, out_hbm, idx_v, ones_v, accum,
                ssem):
    c = lax.axis_index("c")
    s = lax.axis_index("s")
    wid = s * NC + c
    r0 = s * ROWS_PER_TILE
    pltpu.sync_copy(zeros_hbm, accum.at[pl.ds(r0, ROWS_PER_TILE)])
    pltpu.sync_copy(dst_hbm.at[pl.ds(wid * NCHUNK, NCHUNK)], idx_v)
    pltpu.sync_copy(ones_hbm, ones_v)
    plsc.subcore_barrier()

    def body(g, carry):
        for b in range(8):
            pltpu.async_copy(ones_v, accum.at[idx_v.at[g * 8 + b]], ssem,
                             add=True)
        return carry

    lax.fori_loop(0, NCHUNK // 8, body, 0)
    for _ in range(NCHUNK):
        pltpu.make_async_copy(ones_hbm, ones_v, ssem).wait()
    plsc.subcore_barrier()
    pltpu.sync_copy(accum.at[pl.ds(r0, ROWS_PER_TILE)],
                    out_hbm.at[c].at[pl.ds(r0, ROWS_PER_TILE)])


def _make_agg(w):
    kd = 4 if w == 64 else 8
    ngroup = NCHUNK // kd

    @functools.partial(
        pl.kernel,
        out_type=jax.ShapeDtypeStruct((NC, N16, w), jnp.float32),
        mesh=_sc_mesh(),
        compiler_params=_SC_PARAMS,
        scratch_types=[
            pltpu.VMEM((NCHUNK, CH), jnp.int32),
            pltpu.VMEM((NCHUNK, CH), jnp.int32),
            pltpu.VMEM((2, kd, CH, w), jnp.float32),
            pltpu.VMEM_SHARED((N16, w), jnp.float32),
            pltpu.SemaphoreType.DMA,
            pltpu.SemaphoreType.DMA,
        ],
    )
    def _agg(yw_hbm, src_hbm, dst_hbm, zeros_hbm, out_hbm,
             src_v, dst_v, rows_v, accum, gsem, ssem):
        c = lax.axis_index("c")
        s = lax.axis_index("s")
        wid = s * NC + c
        r0 = s * ROWS_PER_TILE
        pltpu.sync_copy(zeros_hbm, accum.at[pl.ds(r0, ROWS_PER_TILE)])
        pltpu.sync_copy(src_hbm.at[pl.ds(wid * NCHUNK, NCHUNK)], src_v)
        pltpu.sync_copy(dst_hbm.at[pl.ds(wid * NCHUNK, NCHUNK)], dst_v)
        plsc.subcore_barrier()

        def fire_gathers(g, p):
            for b in range(kd):
                pltpu.async_copy(yw_hbm.at[src_v.at[g * kd + b]],
                                 rows_v.at[p, b], gsem)

        def fire_scatters(g, p):
            for b in range(kd):
                pltpu.async_copy(rows_v.at[p, b],
                                 accum.at[dst_v.at[g * kd + b]],
                                 ssem, add=True)

        def drain(sem, p):
            for b in range(kd):
                pltpu.make_async_copy(yw_hbm.at[pl.ds(0, CH)],
                                      rows_v.at[p, b], sem).wait()

        def step(g, cur, oth):
            drain(ssem, oth)
            fire_gathers(g + 1, oth)
            drain(gsem, cur)
            fire_scatters(g, cur)

        fire_gathers(0, 0)
        fire_gathers(1, 1)
        drain(gsem, 0)
        fire_scatters(0, 0)

        def body(i, carry):
            step(2 * i + 1, 1, 0)
            step(2 * i + 2, 0, 1)
            return carry

        lax.fori_loop(0, (ngroup - 2) // 2, body, 0)
        drain(gsem, 1)
        fire_scatters(ngroup - 1, 1)
        drain(ssem, 0)
        drain(ssem, 1)
        plsc.subcore_barrier()
        pltpu.sync_copy(accum.at[pl.ds(r0, ROWS_PER_TILE)],
                        out_hbm.at[c].at[pl.ds(r0, ROWS_PER_TILE)])

    return _agg


_agg64 = _make_agg(64)
_agg32 = _make_agg(32)
_agg16 = _make_agg(16)
_AGG = {64: _agg64, 32: _agg32, 16: _agg16}


def _counts_pair(dstp, ones, zeros16):
    return _deg_kernel(dstp, ones, zeros16)


def _agg_pair(yw, srcp, dstp, zeros):
    return _AGG[yw.shape[1]](yw, srcp, dstp, zeros)


def kernel(x, edge_index, W1, b1, W_skip02, b_skip02, W2, b2, W_skip13,
           b_skip13, W3, b3, W_lin, b_lin):
    npad = NW * NCHUNK * CH - E
    pad_src = (jnp.arange(npad, dtype=jnp.int32) * 13) % N
    pad_dst = N + (jnp.arange(npad, dtype=jnp.int32) % (N16 - N))
    srcp = _pad_edges(edge_index[0], pad_src)
    dstp = _pad_edges(edge_index[1], pad_dst)
    ones = jnp.ones((CH, 16), jnp.float32)
    zeros16 = jnp.zeros((ROWS_PER_TILE, 16), jnp.float32)
    zeros64 = jnp.zeros((ROWS_PER_TILE, 64), jnp.float32)
    zeros32 = jnp.zeros((ROWS_PER_TILE, 32), jnp.float32)
    b1r = b1.reshape(1, -1)
    b2r = b2.reshape(1, -1)
    b3r = b3.reshape(1, -1)
    bsk02 = b_skip02.reshape(1, -1)
    bsk13 = b_skip13.reshape(1, -1)
    blr = b_lin.reshape(1, -1)

    counts = _counts_pair(dstp, ones, zeros16)

    xw1, skip02 = _tc_call(
        _b0_body,
        (jax.ShapeDtypeStruct((N, 64), jnp.float32),
         jax.ShapeDtypeStruct((N, 32), jnp.float32)),
        [_row_spec(128), _full_spec(128, 64),
         _full_spec(128, 32), _full_spec(1, 32)],
        [_row_spec(64), _row_spec(32)],
    )(x, W1, W_skip02, bsk02)

    yw1, dis = _tc_call(
        _b1_body,
        (jax.ShapeDtypeStruct((N, 64), jnp.float32),
         jax.ShapeDtypeStruct((GRID, 1, BN), jnp.float32)),
        [_row_spec(64), _pair_spec(16)],
        [_row_spec(64), _vec_spec()],
    )(xw1, counts)

    s1 = _agg_pair(yw1, srcp, dstp, zeros64)

    yw2, skip13 = _tc_call(
        _b2_body,
        (jax.ShapeDtypeStruct((N, 32), jnp.float32),
         jax.ShapeDtypeStruct((N, 16), jnp.float32)),
        [_pair_spec(64), _row_spec(64), _vec_spec(), _full_spec(64, 32),
         _full_spec(1, 64), _full_spec(64, 16), _full_spec(1, 16)],
        [_row_spec(32), _row_spec(16)],
    )(s1, yw1, dis, W2, b1r, W_skip13, bsk13)

    s2 = _agg_pair(yw2, srcp, dstp, zeros32)

    yw3 = _tc_call(
        _b3_body,
        jax.ShapeDtypeStruct((N, 16), jnp.float32),
        [_pair_spec(32), _row_spec(32), _row_spec(32), _vec_spec(),
         _full_spec(32, 16), _full_spec(1, 32)],
        _row_spec(16),
    )(s2, yw2, skip02, dis, W3, b2r)

    s3 = _agg_pair(yw3, srcp, dstp, zeros16)

    out = _tc_call(
        _b4_body,
        jax.ShapeDtypeStruct((N, 1), jnp.float32),
        [_pair_spec(16), _row_spec(16), _row_spec(16), _vec_spec(),
         _full_spec(16, 1), _full_spec(1, 16), _full_spec(1, 1)],
        _row_spec(1),
    )(s3, yw3, skip13, dis, W_lin, b3r, blr)

    return out


# speedup vs baseline: 45.1682x; 1.0006x over previous
import functools

import jax
import jax.numpy as jnp
from jax import lax
from jax.experimental import pallas as pl
from jax.experimental.pallas import tpu as pltpu
from jax.experimental.pallas import tpu_sc as plsc

N = 10000
E = 320000
BN = 2000
GRID = N // BN

NC = 2
NS = 16
NW = NC * NS
CH = 128
PERW = E // NW
NCHUNK = 80
PERW_PAD = NCHUNK * CH
N16 = 10112
ROWS_PER_TILE = N16 // NS


def _tc_call(body, out_shapes, in_specs, out_specs):
    return pl.pallas_call(
        body,
        grid=(GRID,),
        in_specs=in_specs,
        out_specs=out_specs,
        out_shape=out_shapes,
    )


def _row_spec(w):
    return pl.BlockSpec((BN, w), lambda i: (i, 0))


def _pair_spec(w):
    return pl.BlockSpec((2, BN, w), lambda i: (0, i, 0))


def _full_spec(r, c):
    return pl.BlockSpec((r, c), lambda i: (0, 0))


def _vec_spec():
    return pl.BlockSpec((1, 1, BN), lambda i: (i, 0, 0))


def _dis_block(dis_ref):
    return dis_ref[0, 0, :]


def _dis_from_counts(c_ref):
    cb = c_ref[...]
    deg = 1.0 + cb[0, :, 0] + cb[1, :, 0]
    return lax.rsqrt(deg)


def _b0_body(x_ref, w1_ref, wsk_ref, bsk_ref, xw1_ref, skip02_ref):
    xb = x_ref[...]
    xw1_ref[...] = jnp.dot(xb, w1_ref[...], preferred_element_type=jnp.float32)
    skip02_ref[...] = jnp.dot(xb, wsk_ref[...], preferred_element_type=jnp.float32) + bsk_ref[...]


def _b1_body(xw1_ref, c_ref, yw1_ref, dis_ref):
    dis = _dis_from_counts(c_ref)
    yw1_ref[...] = dis[:, None] * xw1_ref[...]
    dis_ref[0, 0, :] = dis


def _b2_body(s1_ref, yw1_ref, dis_ref, w2_ref, b1_ref, wsk_ref, bsk_ref,
             yw2_ref, skip13_ref):
    dis = _dis_block(dis_ref)
    s = s1_ref[0] + s1_ref[1] + yw1_ref[...]
    x1 = jax.nn.relu(dis[:, None] * s + b1_ref[...])
    xw2 = jnp.dot(x1, w2_ref[...], preferred_element_type=jnp.float32)
    yw2_ref[...] = dis[:, None] * xw2
    skip13_ref[...] = jnp.dot(x1, wsk_ref[...], preferred_element_type=jnp.float32) + bsk_ref[...]


def _b3_body(s2_ref, yw2_ref, skip02_ref, dis_ref, w3_ref, b2_ref, yw3_ref):
    dis = _dis_block(dis_ref)
    s = s2_ref[0] + s2_ref[1] + yw2_ref[...]
    x2 = jax.nn.relu(dis[:, None] * s + b2_ref[...] + skip02_ref[...])
    xw3 = jnp.dot(x2, w3_ref[...], preferred_element_type=jnp.float32)
    yw3_ref[...] = dis[:, None] * xw3


def _b4_body(s3_ref, yw3_ref, skip13_ref, dis_ref, wl_ref, b3_ref, bl_ref, out_ref):
    dis = _dis_block(dis_ref)
    s = s3_ref[0] + s3_ref[1] + yw3_ref[...]
    x3 = jax.nn.relu(dis[:, None] * s + b3_ref[...] + skip13_ref[...])
    z = jnp.dot(x3, wl_ref[...], preferred_element_type=jnp.float32) + bl_ref[...]
    out_ref[...] = jax.nn.sigmoid(z)


def _sc_mesh():
    return plsc.VectorSubcoreMesh(core_axis_name="c", subcore_axis_name="s",
                                  num_cores=NC, num_subcores=NS)


_SC_PARAMS = pltpu.CompilerParams(use_tc_tiling_on_sc=False)


def _pad_edges(idx, pad_vals):
    return jnp.concatenate([idx, pad_vals]).reshape(NW * NCHUNK, CH)


@functools.partial(
    pl.kernel,
    out_type=jax.ShapeDtypeStruct((NC, N16, 16), jnp.float32),
    mesh=_sc_mesh(),
    compiler_params=_SC_PARAMS,
    scratch_types=[
        pltpu.VMEM((NCHUNK, CH), jnp.int32),
        pltpu.VMEM((CH, 16), jnp.float32),
        pltpu.VMEM_SHARED((N16, 16), jnp.float32),
        pltpu.SemaphoreType.DMA,
    ],
)
def _deg_kernel(dst_hbm, ones_hbm, zeros_hbm, out_hbm, idx_v, ones_v, accum,
                ssem):
    c = lax.axis_index("c")
    s = lax.axis_index("s")
    wid = s * NC + c
    r0 = s * ROWS_PER_TILE
    pltpu.sync_copy(zeros_hbm, accum.at[pl.ds(r0, ROWS_PER_TILE)])
    pltpu.sync_copy(dst_hbm.at[pl.ds(wid * NCHUNK, NCHUNK)], idx_v)
    pltpu.sync_copy(ones_hbm, ones_v)
    plsc.subcore_barrier()

    def body(g, carry):
        for b in range(8):
            pltpu.async_copy(ones_v, accum.at[idx_v.at[g * 8 + b]], ssem,
                             add=True)
        return carry

    lax.fori_loop(0, NCHUNK // 8, body, 0)
    for _ in range(NCHUNK):
        pltpu.make_async_copy(ones_hbm, ones_v, ssem).wait()
    plsc.subcore_barrier()
    pltpu.sync_copy(accum.at[pl.ds(r0, ROWS_PER_TILE)],
                    out_hbm.at[c].at[pl.ds(r0, ROWS_PER_TILE)])


def _make_agg(w):
    kd = 4 if w == 64 else 8
    ngroup = NCHUNK // kd

    @functools.partial(
        pl.kernel,
        out_type=jax.ShapeDtypeStruct((NC, N16, w), jnp.float32),
        mesh=_sc_mesh(),
        compiler_params=_SC_PARAMS,
        scratch_types=[
            pltpu.VMEM((NCHUNK, CH), jnp.int32),
            pltpu.VMEM((NCHUNK, CH), jnp.int32),
            pltpu.VMEM((2, kd, CH, w), jnp.float32),
            pltpu.VMEM_SHARED((N16, w), jnp.float32),
            pltpu.SemaphoreType.DMA,
            pltpu.SemaphoreType.DMA,
        ],
    )
    def _agg(yw_hbm, src_hbm, dst_hbm, zeros_hbm, out_hbm,
             src_v, dst_v, rows_v, accum, gsem, ssem):
        c = lax.axis_index("c")
        s = lax.axis_index("s")
        wid = s * NC + c
        r0 = s * ROWS_PER_TILE
        pltpu.sync_copy(zeros_hbm, accum.at[pl.ds(r0, ROWS_PER_TILE)])
        pltpu.sync_copy(src_hbm.at[pl.ds(wid * NCHUNK, NCHUNK)], src_v)
        pltpu.sync_copy(dst_hbm.at[pl.ds(wid * NCHUNK, NCHUNK)], dst_v)
        plsc.subcore_barrier()

        def fire_gathers(g, p):
            for b in range(kd):
                pltpu.async_copy(yw_hbm.at[src_v.at[g * kd + b]],
                                 rows_v.at[p, b], gsem)

        def fire_scatters(g, p):
            for b in range(kd):
                pltpu.async_copy(rows_v.at[p, b],
                                 accum.at[dst_v.at[g * kd + b]],
                                 ssem, add=True)

        def drain(sem, p):
            for b in range(kd):
                pltpu.make_async_copy(yw_hbm.at[pl.ds(0, CH)],
                                      rows_v.at[p, b], sem).wait()

        def step(g, cur, oth):
            drain(ssem, oth)
            fire_gathers(g + 1, oth)
            drain(gsem, cur)
            fire_scatters(g, cur)

        fire_gathers(0, 0)
        fire_gathers(1, 1)
        drain(gsem, 0)
        fire_scatters(0, 0)

        def body(i, carry):
            step(2 * i + 1, 1, 0)
            step(2 * i + 2, 0, 1)
            return carry

        lax.fori_loop(0, (ngroup - 2) // 2, body, 0)
        drain(gsem, 1)
        fire_scatters(ngroup - 1, 1)
        drain(ssem, 0)
        drain(ssem, 1)
        plsc.subcore_barrier()
        pltpu.sync_copy(accum.at[pl.ds(r0, ROWS_PER_TILE)],
                        out_hbm.at[c].at[pl.ds(r0, ROWS_PER_TILE)])

    return _agg


_agg64 = _make_agg(64)
_agg32 = _make_agg(32)
_agg16 = _make_agg(16)
_AGG = {64: _agg64, 32: _agg32, 16: _agg16}


def _counts_pair(dstp, ones, zeros16):
    return _deg_kernel(dstp, ones, zeros16)


def _agg_pair(yw, srcp, dstp, zeros):
    return _AGG[yw.shape[1]](yw, srcp, dstp, zeros)


def kernel(x, edge_index, W1, b1, W_skip02, b_skip02, W2, b2, W_skip13,
           b_skip13, W3, b3, W_lin, b_lin):
    npad = NW * NCHUNK * CH - E
    pad_src = (jnp.arange(npad, dtype=jnp.int32) * 13) % N
    pad_dst = N + (jnp.arange(npad, dtype=jnp.int32) % (N16 - N))
    srcp = _pad_edges(edge_index[0], pad_src)
    dstp = _pad_edges(edge_index[1], pad_dst)
    ones = jnp.ones((CH, 16), jnp.float32)
    zeros16 = jnp.zeros((ROWS_PER_TILE, 16), jnp.float32)
    zeros64 = jnp.zeros((ROWS_PER_TILE, 64), jnp.float32)
    zeros32 = jnp.zeros((ROWS_PER_TILE, 32), jnp.float32)
    b1r = b1.reshape(1, -1)
    b2r = b2.reshape(1, -1)
    b3r = b3.reshape(1, -1)
    bsk02 = b_skip02.reshape(1, -1)
    bsk13 = b_skip13.reshape(1, -1)
    blr = b_lin.reshape(1, -1)

    counts = _counts_pair(dstp, ones, zeros16)

    xw1, skip02 = _tc_call(
        _b0_body,
        (jax.ShapeDtypeStruct((N, 64), jnp.float32),
         jax.ShapeDtypeStruct((N, 32), jnp.float32)),
        [_row_spec(128), _full_spec(128, 64),
         _full_spec(128, 32), _full_spec(1, 32)],
        [_row_spec(64), _row_spec(32)],
    )(x, W1, W_skip02, bsk02)

    yw1, dis = _tc_call(
        _b1_body,
        (jax.ShapeDtypeStruct((N, 64), jnp.float32),
         jax.ShapeDtypeStruct((GRID, 1, BN), jnp.float32)),
        [_row_spec(64), _pair_spec(16)],
        [_row_spec(64), _vec_spec()],
    )(xw1, counts)

    s1 = _agg_pair(yw1, srcp, dstp, zeros64)

    yw2, skip13 = _tc_call(
        _b2_body,
        (jax.ShapeDtypeStruct((N, 32), jnp.float32),
         jax.ShapeDtypeStruct((N, 16), jnp.float32)),
        [_pair_spec(64), _row_spec(64), _vec_spec(),
         _full_spec(64, 32), _full_spec(1, 64), _full_spec(64, 16),
         _full_spec(1, 16)],
        [_row_spec(32), _row_spec(16)],
    )(s1, yw1, dis, W2, b1r, W_skip13, bsk13)

    s2 = _agg_pair(yw2, srcp, dstp, zeros32)

    yw3 = _tc_call(
        _b3_body,
        jax.ShapeDtypeStruct((N, 16), jnp.float32),
        [_pair_spec(32), _row_spec(32), _row_spec(32), _vec_spec(),
         _full_spec(32, 16), _full_spec(1, 32)],
        _row_spec(16),
    )(s2, yw2, skip02, dis, W3, b2r)

    s3 = _agg_pair(yw3, srcp, dstp, zeros16)

    out = _tc_call(
        _b4_body,
        jax.ShapeDtypeStruct((N, 1), jnp.float32),
        [_pair_spec(16), _row_spec(16), _row_spec(16), _vec_spec(),
         _full_spec(16, 1), _full_spec(1, 16), _full_spec(1, 1)],
        _row_spec(1),
    )(s3, yw3, skip13, dis, W_lin, b3r, blr)

    return out
